# baseline (device time: 106842 ns/iter reference)
import jax
import jax.numpy as jnp
from jax import lax
from jax.experimental import pallas as pl
from jax.experimental.pallas import tpu as pltpu

N_X = 2
CH = 512
NQ = 4
D_X = 688
D_Y = 688
D_Z = 672


def kernel(x):
    m, n = x.shape
    quarter = m // 4
    n_conv = m // CH

    def body(x_hbm, out_hbm, own_bf16, miss_bf16, stag, stag_sem,
             x_send, x_recv, x2_send, x2_recv,
             yf_send, yf_recv, zf_send, zf_recv,
             yh_send, yh_recv, zh_send, zh_recv, own_sem, miss_sem):
        my_x = lax.axis_index("x")
        my_y = lax.axis_index("y")
        my_z = lax.axis_index("z")
        x_peer = (1 - my_x, my_y, my_z)
        y_peer = (my_x, 1 - my_y, my_z)
        z_peer = (my_x, my_y, 1 - my_z)

        miss = (1 - my_x) * m
        q_me = 2 * my_y + my_z
        q_yn = 2 * (1 - my_y) + my_z
        q_zn = 2 * my_y + (1 - my_z)
        q_d = 2 * (1 - my_y) + (1 - my_z)
        qoff = q_me * quarter
        doff = q_d * quarter

        xch = [(0, 256), (256, 512), (768, 640), (1408, 640)]
        conv_jobs = (
            [(qoff + o, r) for o, r in xch]
            + [(doff + j * CH, CH) for j in range(NQ)]
            + [(q_yn * quarter + j * CH, CH) for j in range(NQ)]
            + [(q_zn * quarter + j * CH, CH) for j in range(NQ)]
        )

        def stage_start(i):
            off, rows = conv_jobs[i]
            cp = pltpu.make_async_copy(
                x_hbm.at[pl.ds(off, rows), :],
                stag.at[i % 2, pl.ds(0, rows), :], stag_sem.at[i % 2],
            )
            cp.start()
            return cp

        def do_convert(i):
            if i + 1 < n_conv:
                stage_start(i + 1)
            off, rows = conv_jobs[i]
            pltpu.make_async_copy(
                x_hbm.at[pl.ds(off, rows), :],
                stag.at[i % 2, pl.ds(0, rows), :], stag_sem.at[i % 2],
            ).wait()
            own_bf16[pl.ds(off, rows), :] = (
                stag[i % 2, 0:rows, :].astype(jnp.bfloat16))

        def remote(src_ref, dst_ref, ssem, rsem, peer):
            r = pltpu.make_async_remote_copy(
                src_ref=src_ref, dst_ref=dst_ref, send_sem=ssem,
                recv_sem=rsem, device_id=peer,
                device_id_type=pl.DeviceIdType.MESH,
            )
            r.start()
            return r

        barrier_sem = pltpu.get_barrier_semaphore()
        for nbr in [x_peer, y_peer, z_peer]:
            pl.semaphore_signal(
                barrier_sem, inc=1, device_id=nbr,
                device_id_type=pl.DeviceIdType.MESH,
            )
        stage_start(0)
        pl.semaphore_wait(barrier_sem, 3)

        x_rdmas = []
        for i in range(NQ):
            do_convert(i)
            o, r = xch[i]
            x_rdmas.append(remote(
                own_bf16.at[pl.ds(qoff + o, r), :],
                miss_bf16.at[pl.ds(qoff + o, r), :],
                x_send.at[i], x_recv.at[i], x_peer,
            ))
        do_convert(NQ)
        do_convert(NQ + 1)
        x2_rdma = remote(
            own_bf16.at[pl.ds(doff, D_X), :],
            miss_bf16.at[pl.ds(doff, D_X), :],
            x2_send, x2_recv, x_peer,
        )

        yf_rdmas, zf_rdmas = [], []
        for i in range(NQ):
            x_rdmas[i].wait_recv()
            o, r = xch[i]
            src = miss_bf16.at[pl.ds(qoff + o, r), :]
            yf_rdmas.append(remote(
                src, miss_bf16.at[pl.ds(qoff + o, r), :],
                yf_send.at[i], yf_recv.at[i], y_peer,
            ))
            zf_rdmas.append(remote(
                src, miss_bf16.at[pl.ds(qoff + o, r), :],
                zf_send.at[i], zf_recv.at[i], z_peer,
            ))
            do_convert(NQ + 2 + i)

        miss_dmas = []
        cp = pltpu.make_async_copy(
            miss_bf16.at[pl.ds(qoff, quarter), :],
            out_hbm.at[pl.ds(miss + qoff, quarter), :], miss_sem.at[0],
        )
        cp.start()
        miss_dmas.append(cp)

        yh_rdma = zh_rdma = None
        ci = NQ + 2 + NQ
        for i in range(NQ):
            zf_rdmas[i].wait_recv()
            if i == 2:
                yh_rdma = remote(
                    miss_bf16.at[pl.ds(q_zn * quarter + D_X, D_Y), :],
                    miss_bf16.at[pl.ds(q_zn * quarter + D_X, D_Y), :],
                    yh_send, yh_recv, y_peer,
                )
            yf_rdmas[i].wait_recv()
            if i == 3:
                zh_rdma = remote(
                    miss_bf16.at[pl.ds(q_yn * quarter + D_X + D_Y, D_Z), :],
                    miss_bf16.at[pl.ds(q_yn * quarter + D_X + D_Y, D_Z), :],
                    zh_send, zh_recv, z_peer,
                )
            while ci < n_conv and ci < NQ + 2 + NQ + 2 * (i + 1):
                do_convert(ci)
                ci += 1

        own_cp = pltpu.make_async_copy(
            own_bf16, out_hbm.at[pl.ds(my_x * m, m), :], own_sem,
        )
        own_cp.start()
        for si, q in ((1, q_zn), (2, q_yn)):
            cp = pltpu.make_async_copy(
                miss_bf16.at[pl.ds(q * quarter, quarter), :],
                out_hbm.at[pl.ds(miss + q * quarter, quarter), :],
                miss_sem.at[si],
            )
            cp.start()
            miss_dmas.append(cp)

        x2_rdma.wait_recv()
        yh_rdma.wait_recv()
        zh_rdma.wait_recv()
        cp = pltpu.make_async_copy(
            miss_bf16.at[pl.ds(doff, quarter), :],
            out_hbm.at[pl.ds(miss + doff, quarter), :],
            miss_sem.at[3],
        )
        cp.start()
        miss_dmas.append(cp)

        for r in x_rdmas + yf_rdmas + zf_rdmas + [x2_rdma, yh_rdma, zh_rdma]:
            r.wait_send()
        own_cp.wait()
        for cp in miss_dmas:
            cp.wait()

    return pl.pallas_call(
        body,
        out_shape=jax.ShapeDtypeStruct((N_X * m, n), jnp.bfloat16),
        in_specs=[pl.BlockSpec(memory_space=pl.ANY)],
        out_specs=pl.BlockSpec(memory_space=pl.ANY),
        scratch_shapes=[
            pltpu.VMEM((m, n), jnp.bfloat16),
            pltpu.VMEM((m, n), jnp.bfloat16),
            pltpu.VMEM((2, 640, n), jnp.float32),
            pltpu.SemaphoreType.DMA((2,)),
            pltpu.SemaphoreType.DMA((NQ,)),
            pltpu.SemaphoreType.DMA((NQ,)),
            pltpu.SemaphoreType.DMA,
            pltpu.SemaphoreType.DMA,
            pltpu.SemaphoreType.DMA((NQ,)),
            pltpu.SemaphoreType.DMA((NQ,)),
            pltpu.SemaphoreType.DMA((NQ,)),
            pltpu.SemaphoreType.DMA((NQ,)),
            pltpu.SemaphoreType.DMA,
            pltpu.SemaphoreType.DMA,
            pltpu.SemaphoreType.DMA,
            pltpu.SemaphoreType.DMA,
            pltpu.SemaphoreType.DMA,
            pltpu.SemaphoreType.DMA((4,)),
        ],
        compiler_params=pltpu.CompilerParams(
            collective_id=0,
            vmem_limit_bytes=48 * 1024 * 1024,
        ),
    )(x)


# device time: 104744 ns/iter; 1.0200x vs baseline; 1.0200x over previous
import jax
import jax.numpy as jnp
from jax import lax
from jax.experimental import pallas as pl
from jax.experimental.pallas import tpu as pltpu

N_X = 2
CH = 512
NQ = 4
D_X = 688
D_Y = 688
D_Z = 672


def kernel(x):
    m, n = x.shape
    quarter = m // 4
    n_conv = m // CH

    def body(x_hbm, out_hbm, own_bf16, miss_bf16, stag, stag_sem,
             x_send, x_recv, x2_send, x2_recv,
             yf_send, yf_recv, zf_send, zf_recv,
             yh_send, yh_recv, zh_send, zh_recv, own_sem, miss_sem):
        my_x = lax.axis_index("x")
        my_y = lax.axis_index("y")
        my_z = lax.axis_index("z")
        x_peer = (1 - my_x, my_y, my_z)
        y_peer = (my_x, 1 - my_y, my_z)
        z_peer = (my_x, my_y, 1 - my_z)

        miss = (1 - my_x) * m
        q_me = 2 * my_y + my_z
        q_yn = 2 * (1 - my_y) + my_z
        q_zn = 2 * my_y + (1 - my_z)
        q_d = 2 * (1 - my_y) + (1 - my_z)
        qoff = q_me * quarter
        doff = q_d * quarter

        xch = [(j * CH, CH) for j in range(NQ)]
        conv_jobs = (
            [(qoff + o, r) for o, r in xch]
            + [(doff + j * CH, CH) for j in range(NQ)]
            + [(q_yn * quarter + j * CH, CH) for j in range(NQ)]
            + [(q_zn * quarter + j * CH, CH) for j in range(NQ)]
        )

        def stage_start(i):
            off, rows = conv_jobs[i]
            cp = pltpu.make_async_copy(
                x_hbm.at[pl.ds(off, rows), :],
                stag.at[i % 2, pl.ds(0, rows), :], stag_sem.at[i % 2],
            )
            cp.start()
            return cp

        def do_convert(i):
            if i + 1 < n_conv:
                stage_start(i + 1)
            off, rows = conv_jobs[i]
            pltpu.make_async_copy(
                x_hbm.at[pl.ds(off, rows), :],
                stag.at[i % 2, pl.ds(0, rows), :], stag_sem.at[i % 2],
            ).wait()
            own_bf16[pl.ds(off, rows), :] = (
                stag[i % 2, 0:rows, :].astype(jnp.bfloat16))

        def remote(src_ref, dst_ref, ssem, rsem, peer):
            r = pltpu.make_async_remote_copy(
                src_ref=src_ref, dst_ref=dst_ref, send_sem=ssem,
                recv_sem=rsem, device_id=peer,
                device_id_type=pl.DeviceIdType.MESH,
            )
            r.start()
            return r

        barrier_sem = pltpu.get_barrier_semaphore()
        for nbr in [x_peer, y_peer, z_peer]:
            pl.semaphore_signal(
                barrier_sem, inc=1, device_id=nbr,
                device_id_type=pl.DeviceIdType.MESH,
            )
        pl.semaphore_wait(barrier_sem, 3)
        stage_start(0)

        x_rdmas = []
        for i in range(NQ):
            do_convert(i)
            o, r = xch[i]
            x_rdmas.append(remote(
                own_bf16.at[pl.ds(qoff + o, r), :],
                miss_bf16.at[pl.ds(qoff + o, r), :],
                x_send.at[i], x_recv.at[i], x_peer,
            ))
        do_convert(NQ)
        do_convert(NQ + 1)
        x2_rdma = remote(
            own_bf16.at[pl.ds(doff, D_X), :],
            miss_bf16.at[pl.ds(doff, D_X), :],
            x2_send, x2_recv, x_peer,
        )

        yf_rdmas, zf_rdmas = [], []
        for i in range(NQ):
            x_rdmas[i].wait_recv()
            o, r = xch[i]
            src = miss_bf16.at[pl.ds(qoff + o, r), :]
            yf_rdmas.append(remote(
                src, miss_bf16.at[pl.ds(qoff + o, r), :],
                yf_send.at[i], yf_recv.at[i], y_peer,
            ))
            zf_rdmas.append(remote(
                src, miss_bf16.at[pl.ds(qoff + o, r), :],
                zf_send.at[i], zf_recv.at[i], z_peer,
            ))
            do_convert(NQ + 2 + i)

        miss_dmas = []
        cp = pltpu.make_async_copy(
            miss_bf16.at[pl.ds(qoff, quarter), :],
            out_hbm.at[pl.ds(miss + qoff, quarter), :], miss_sem.at[0],
        )
        cp.start()
        miss_dmas.append(cp)

        yh_rdma = zh_rdma = None
        ci = NQ + 2 + NQ
        for i in range(NQ):
            zf_rdmas[i].wait_recv()
            if i == 2:
                yh_rdma = remote(
                    miss_bf16.at[pl.ds(q_zn * quarter + D_X, D_Y), :],
                    miss_bf16.at[pl.ds(q_zn * quarter + D_X, D_Y), :],
                    yh_send, yh_recv, y_peer,
                )
            yf_rdmas[i].wait_recv()
            if i == 3:
                zh_rdma = remote(
                    miss_bf16.at[pl.ds(q_yn * quarter + D_X + D_Y, D_Z), :],
                    miss_bf16.at[pl.ds(q_yn * quarter + D_X + D_Y, D_Z), :],
                    zh_send, zh_recv, z_peer,
                )
            while ci < n_conv and ci < NQ + 2 + NQ + 2 * (i + 1):
                do_convert(ci)
                ci += 1

        own_cp = pltpu.make_async_copy(
            own_bf16, out_hbm.at[pl.ds(my_x * m, m), :], own_sem,
        )
        own_cp.start()
        for si, q in ((1, q_zn), (2, q_yn)):
            cp = pltpu.make_async_copy(
                miss_bf16.at[pl.ds(q * quarter, quarter), :],
                out_hbm.at[pl.ds(miss + q * quarter, quarter), :],
                miss_sem.at[si],
            )
            cp.start()
            miss_dmas.append(cp)

        x2_rdma.wait_recv()
        yh_rdma.wait_recv()
        zh_rdma.wait_recv()
        cp = pltpu.make_async_copy(
            miss_bf16.at[pl.ds(doff, quarter), :],
            out_hbm.at[pl.ds(miss + doff, quarter), :],
            miss_sem.at[3],
        )
        cp.start()
        miss_dmas.append(cp)

        for r in x_rdmas + yf_rdmas + zf_rdmas + [x2_rdma, yh_rdma, zh_rdma]:
            r.wait_send()
        own_cp.wait()
        for cp in miss_dmas:
            cp.wait()

    return pl.pallas_call(
        body,
        out_shape=jax.ShapeDtypeStruct((N_X * m, n), jnp.bfloat16),
        in_specs=[pl.BlockSpec(memory_space=pl.ANY)],
        out_specs=pl.BlockSpec(memory_space=pl.ANY),
        scratch_shapes=[
            pltpu.VMEM((m, n), jnp.bfloat16),
            pltpu.VMEM((m, n), jnp.bfloat16),
            pltpu.VMEM((2, CH, n), jnp.float32),
            pltpu.SemaphoreType.DMA((2,)),
            pltpu.SemaphoreType.DMA((NQ,)),
            pltpu.SemaphoreType.DMA((NQ,)),
            pltpu.SemaphoreType.DMA,
            pltpu.SemaphoreType.DMA,
            pltpu.SemaphoreType.DMA((NQ,)),
            pltpu.SemaphoreType.DMA((NQ,)),
            pltpu.SemaphoreType.DMA((NQ,)),
            pltpu.SemaphoreType.DMA((NQ,)),
            pltpu.SemaphoreType.DMA,
            pltpu.SemaphoreType.DMA,
            pltpu.SemaphoreType.DMA,
            pltpu.SemaphoreType.DMA,
            pltpu.SemaphoreType.DMA,
            pltpu.SemaphoreType.DMA((4,)),
        ],
        compiler_params=pltpu.CompilerParams(
            collective_id=0,
            vmem_limit_bytes=48 * 1024 * 1024,
        ),
    )(x)
